# baseline (speedup 1.0000x reference)
import functools

import jax
import jax.numpy as jnp
from jax import lax
from jax.experimental import pallas as pl
from jax.experimental.pallas import tpu as pltpu
from jax.experimental.pallas import tpu_sc as plsc

N = 10000
E = 320000
D = 128

MIN_NORM = 1e-15
MAXNORM = 1.0 - 4e-3

NC, NS, L = 2, 16, 16
NW = NC * NS
CHUNK = 128
NCHUNKS = 2560
EPAD = NCHUNKS * CHUNK
CPT = NCHUNKS // NW
NPAD = 10112
RPT = NPAD // NS

BLK = 2000
GRID = N // BLK


def _rownorm2(x):
    return jnp.sum(x * x, axis=-1, keepdims=True)


def _expmap0(u):
    n = jnp.maximum(jnp.sqrt(_rownorm2(u)), MIN_NORM)
    return jnp.tanh(n) * u / n


def _artanh(x):
    x = jnp.clip(x, -1.0 + 1e-7, 1.0 - 1e-7)
    return 0.5 * jnp.log((1.0 + x) / (1.0 - x))


def _logmap0(p):
    n = jnp.maximum(jnp.sqrt(_rownorm2(p)), MIN_NORM)
    return _artanh(n) * p / n


def _proj(x):
    n = jnp.maximum(jnp.sqrt(_rownorm2(x)), MIN_NORM)
    return jnp.where(n > MAXNORM, x / n * MAXNORM, x)


def _mobius_add(x, y):
    x2 = _rownorm2(x)
    y2 = _rownorm2(y)
    xy = jnp.sum(x * y, axis=-1, keepdims=True)
    num = (1.0 + 2.0 * xy + y2) * x + (1.0 - x2) * y
    denom = 1.0 + 2.0 * xy + x2 * y2
    return num / jnp.maximum(denom, MIN_NORM)


def _mobius_matvec(Wt, x):
    x2s = _rownorm2(x)
    xn = jnp.maximum(jnp.sqrt(x2s), MIN_NORM)
    mx = jnp.dot(x, Wt, preferred_element_type=jnp.float32)
    mx2s = _rownorm2(mx)
    mxn = jnp.maximum(jnp.sqrt(mx2s), MIN_NORM)
    res = jnp.tanh(mxn / xn * _artanh(xn)) * mx / mxn
    return jnp.where(mx2s == 0.0, jnp.zeros_like(res), res)


def _hyp_linear(x, Wt, b):
    res = _proj(_mobius_matvec(Wt, x))
    hyp_bias = _proj(_expmap0(b))
    return _proj(_mobius_add(res, hyp_bias))


def _tc_pre_body(x_ref, w_ref, b_ref, o_ref):
    xh = _proj(_expmap0(x_ref[...]))
    h = _hyp_linear(xh, w_ref[...], b_ref[...])
    o_ref[...] = _logmap0(h)


def _tc_mid_body(p_ref, w_ref, b_ref, o_ref):
    s = p_ref[0] + p_ref[1]
    h = _proj(_expmap0(s))
    h = _proj(_expmap0(jnp.maximum(_logmap0(h), 0.0)))
    h = _hyp_linear(h, w_ref[...], b_ref[...])
    o_ref[...] = _logmap0(h)


def _tc_post_body(p_ref, o_ref):
    s = p_ref[0] + p_ref[1]
    h = _proj(_expmap0(s))
    o_ref[...] = _proj(_expmap0(jnp.maximum(_logmap0(h), 0.0)))


_row_spec = pl.BlockSpec((BLK, D), lambda i: (i, 0))
_par_spec = pl.BlockSpec((2, BLK, D), lambda i: (0, i, 0))
_w_spec = pl.BlockSpec((D, D), lambda i: (0, 0))
_b_spec = pl.BlockSpec((1, D), lambda i: (0, 0))
_out_sd = jax.ShapeDtypeStruct((N, D), jnp.float32)

_tc_pre = pl.pallas_call(
    _tc_pre_body, grid=(GRID,),
    in_specs=[_row_spec, _w_spec, _b_spec], out_specs=_row_spec,
    out_shape=_out_sd)

_tc_mid = pl.pallas_call(
    _tc_mid_body, grid=(GRID,),
    in_specs=[_par_spec, _w_spec, _b_spec], out_specs=_row_spec,
    out_shape=_out_sd)

_tc_post = pl.pallas_call(
    _tc_post_body, grid=(GRID,),
    in_specs=[_par_spec], out_specs=_row_spec,
    out_shape=_out_sd)


NBUF = 2
GCH = 16
CPT0 = 5 * GCH
CPT1 = 5 * GCH


def _run_edges(t_hbm, srcr_hbm, dstr_hbm, src_v, dst_v, rows_v, acc_sh,
               gsems, ssems, base, ngroups):
    for g in range(ngroups):
        pltpu.sync_copy(srcr_hbm.at[pl.ds(base + g * GCH, GCH)], src_v)
        pltpu.sync_copy(dstr_hbm.at[pl.ds(base + g * GCH, GCH)], dst_v)
        for b in range(NBUF):
            pltpu.async_copy(t_hbm.at[src_v.at[b]], rows_v.at[b], gsems[b])

        def body(i, carry):
            for b in range(NBUF):
                j = i * NBUF + b
                pltpu.make_async_copy(
                    t_hbm.at[pl.ds(0, CHUNK)], rows_v.at[b], gsems[b]).wait()
                pltpu.async_copy(rows_v.at[b], acc_sh.at[dst_v.at[j]],
                                 ssems[b], add=True)
            for b in range(NBUF):
                j = i * NBUF + b

                @pl.when(j + NBUF < GCH)
                def _():
                    pltpu.make_async_copy(
                        rows_v.at[b], acc_sh.at[pl.ds(0, CHUNK)],
                        ssems[b]).wait()
                    pltpu.async_copy(
                        t_hbm.at[src_v.at[j + NBUF]], rows_v.at[b], gsems[b])
            return carry

        lax.fori_loop(0, GCH // NBUF, body, 0)
        for b in range(NBUF):
            pltpu.make_async_copy(
                rows_v.at[b], acc_sh.at[pl.ds(0, CHUNK)], ssems[b]).wait()


def _sc_agg_body(t_hbm, srcr_hbm, dstr_hbm, zeros_hbm, out_hbm,
                 src_v, dst_v, rows_v, acc_sh, *sems):
    c = lax.axis_index("c")
    s = lax.axis_index("s")
    pltpu.sync_copy(zeros_hbm.at[pl.ds(s * RPT, RPT)],
                    acc_sh.at[pl.ds(s * RPT, RPT)])
    plsc.subcore_barrier()

    gsems, ssems = sems[:NBUF], sems[NBUF:]

    @pl.when(c == 0)
    def _():
        _run_edges(t_hbm, srcr_hbm, dstr_hbm, src_v, dst_v, rows_v, acc_sh,
                   gsems, ssems, s * CPT0, CPT0 // GCH)

    @pl.when(c == 1)
    def _():
        _run_edges(t_hbm, srcr_hbm, dstr_hbm, src_v, dst_v, rows_v, acc_sh,
                   gsems, ssems, NS * CPT0 + s * CPT1, CPT1 // GCH)

    plsc.subcore_barrier()
    pltpu.sync_copy(acc_sh.at[pl.ds(s * RPT, RPT)],
                    out_hbm.at[c].at[pl.ds(s * RPT, RPT)])


@functools.cache
def _sc_agg_call():
    return pl.kernel(
        _sc_agg_body,
        out_type=jax.ShapeDtypeStruct((NC, NPAD, D), jnp.float32),
        mesh=plsc.VectorSubcoreMesh(core_axis_name="c", subcore_axis_name="s"),
        scratch_types=[
            pltpu.VMEM((GCH, CHUNK), jnp.int32),
            pltpu.VMEM((GCH, CHUNK), jnp.int32),
            pltpu.VMEM((NBUF, CHUNK, D), jnp.float32),
            pltpu.VMEM_SHARED((NPAD, D), jnp.float32),
        ] + [pltpu.SemaphoreType.DMA] * (2 * NBUF),
    )


def kernel(x, edge_index, W0, b0, W1, b1):
    src = edge_index[0].astype(jnp.int32)
    dst = edge_index[1].astype(jnp.int32)
    srcr = jnp.concatenate(
        [src, jnp.arange(EPAD - E, dtype=jnp.int32)]).reshape(NCHUNKS, CHUNK)
    dstr = jnp.concatenate(
        [dst, jnp.full((EPAD - E,), N, jnp.int32)]).reshape(NCHUNKS, CHUNK)
    zeros = jnp.zeros((NPAD, D), jnp.float32)
    W0t = W0.T
    W1t = W1.T
    b0r = b0.reshape(1, D)
    b1r = b1.reshape(1, D)

    sc_agg = _sc_agg_call()
    t0 = _tc_pre(x, W0t, b0r)
    p0 = sc_agg(t0, srcr, dstr, zeros)
    t1 = _tc_mid(p0, W1t, b1r)
    p1 = sc_agg(t1, srcr, dstr, zeros)
    return _tc_post(p1)


# speedup vs baseline: 1.2538x; 1.2538x over previous
import functools

import jax
import jax.numpy as jnp
from jax import lax
from jax.experimental import pallas as pl
from jax.experimental.pallas import tpu as pltpu
from jax.experimental.pallas import tpu_sc as plsc

N = 10000
E = 320000
D = 128

MIN_NORM = 1e-15
MAXNORM = 1.0 - 4e-3

NC, NS, L = 2, 16, 16
NW = NC * NS
CHUNK = 128
NCHUNKS = 2560
EPAD = NCHUNKS * CHUNK
CPT = NCHUNKS // NW
NPAD = 10112
RPT = NPAD // NS

BLK = 2000
GRID = N // BLK


def _rownorm2(x):
    return jnp.sum(x * x, axis=-1, keepdims=True)


def _expmap0(u):
    n = jnp.maximum(jnp.sqrt(_rownorm2(u)), MIN_NORM)
    return jnp.tanh(n) * u / n


def _artanh(x):
    x = jnp.clip(x, -1.0 + 1e-7, 1.0 - 1e-7)
    return 0.5 * jnp.log((1.0 + x) / (1.0 - x))


def _logmap0(p):
    n = jnp.maximum(jnp.sqrt(_rownorm2(p)), MIN_NORM)
    return _artanh(n) * p / n


def _proj(x):
    n = jnp.maximum(jnp.sqrt(_rownorm2(x)), MIN_NORM)
    return jnp.where(n > MAXNORM, x / n * MAXNORM, x)


def _mobius_add(x, y):
    x2 = _rownorm2(x)
    y2 = _rownorm2(y)
    xy = jnp.sum(x * y, axis=-1, keepdims=True)
    num = (1.0 + 2.0 * xy + y2) * x + (1.0 - x2) * y
    denom = 1.0 + 2.0 * xy + x2 * y2
    return num / jnp.maximum(denom, MIN_NORM)


def _mobius_matvec(Wt, x):
    x2s = _rownorm2(x)
    xn = jnp.maximum(jnp.sqrt(x2s), MIN_NORM)
    mx = jnp.dot(x, Wt, preferred_element_type=jnp.float32)
    mx2s = _rownorm2(mx)
    mxn = jnp.maximum(jnp.sqrt(mx2s), MIN_NORM)
    res = jnp.tanh(mxn / xn * _artanh(xn)) * mx / mxn
    return jnp.where(mx2s == 0.0, jnp.zeros_like(res), res)


def _hyp_linear(x, Wt, b):
    res = _proj(_mobius_matvec(Wt, x))
    hyp_bias = _proj(_expmap0(b))
    return _proj(_mobius_add(res, hyp_bias))


def _tc_pre_body(x_ref, w_ref, b_ref, o_ref):
    xh = _proj(_expmap0(x_ref[...]))
    h = _hyp_linear(xh, w_ref[...], b_ref[...])
    o_ref[...] = _logmap0(h)


def _tc_mid_body(p_ref, w_ref, b_ref, o_ref):
    s = p_ref[0] + p_ref[1]
    h = _proj(_expmap0(s))
    h = _proj(_expmap0(jnp.maximum(_logmap0(h), 0.0)))
    h = _hyp_linear(h, w_ref[...], b_ref[...])
    o_ref[...] = _logmap0(h)


def _tc_post_body(p_ref, o_ref):
    s = p_ref[0] + p_ref[1]
    h = _proj(_expmap0(s))
    o_ref[...] = _proj(_expmap0(jnp.maximum(_logmap0(h), 0.0)))


_row_spec = pl.BlockSpec((BLK, D), lambda i: (i, 0))
_par_spec = pl.BlockSpec((2, BLK, D), lambda i: (0, i, 0))
_w_spec = pl.BlockSpec((D, D), lambda i: (0, 0))
_b_spec = pl.BlockSpec((1, D), lambda i: (0, 0))
_out_sd = jax.ShapeDtypeStruct((N, D), jnp.float32)

_tc_pre = pl.pallas_call(
    _tc_pre_body, grid=(GRID,),
    in_specs=[_row_spec, _w_spec, _b_spec], out_specs=_row_spec,
    out_shape=_out_sd)

_tc_mid = pl.pallas_call(
    _tc_mid_body, grid=(GRID,),
    in_specs=[_par_spec, _w_spec, _b_spec], out_specs=_row_spec,
    out_shape=_out_sd)

_tc_post = pl.pallas_call(
    _tc_post_body, grid=(GRID,),
    in_specs=[_par_spec], out_specs=_row_spec,
    out_shape=_out_sd)


NBUF = 2
GCH = 40
CPT0 = 2 * GCH
CPT1 = 2 * GCH


def _run_edges(t_hbm, srcr_hbm, dstr_hbm, src_v, dst_v, rows_v, acc_sh,
               gsems, ssems, base, ngroups):
    for g in range(ngroups):
        pltpu.sync_copy(srcr_hbm.at[pl.ds(base + g * GCH, GCH)], src_v)
        pltpu.sync_copy(dstr_hbm.at[pl.ds(base + g * GCH, GCH)], dst_v)
        for b in range(NBUF):
            pltpu.async_copy(t_hbm.at[src_v.at[b]], rows_v.at[b], gsems[b])

        def body(i, carry):
            for b in range(NBUF):
                j = i * NBUF + b
                pltpu.make_async_copy(
                    t_hbm.at[pl.ds(0, CHUNK)], rows_v.at[b], gsems[b]).wait()
                pltpu.sync_copy(rows_v.at[b], acc_sh.at[dst_v.at[j]],
                                add=True)

                @pl.when(j + NBUF < GCH)
                def _():
                    pltpu.async_copy(
                        t_hbm.at[src_v.at[j + NBUF]], rows_v.at[b], gsems[b])
            return carry

        lax.fori_loop(0, GCH // NBUF, body, 0)


def _sc_agg_body(t_hbm, srcr_hbm, dstr_hbm, zeros_hbm, out_hbm,
                 src_v, dst_v, rows_v, acc_sh, *sems):
    c = lax.axis_index("c")
    s = lax.axis_index("s")
    pltpu.sync_copy(zeros_hbm.at[pl.ds(s * RPT, RPT)],
                    acc_sh.at[pl.ds(s * RPT, RPT)])
    plsc.subcore_barrier()

    gsems, ssems = sems[:NBUF], sems[NBUF:]

    @pl.when(c == 0)
    def _():
        _run_edges(t_hbm, srcr_hbm, dstr_hbm, src_v, dst_v, rows_v, acc_sh,
                   gsems, ssems, s * CPT0, CPT0 // GCH)

    @pl.when(c == 1)
    def _():
        _run_edges(t_hbm, srcr_hbm, dstr_hbm, src_v, dst_v, rows_v, acc_sh,
                   gsems, ssems, NS * CPT0 + s * CPT1, CPT1 // GCH)

    plsc.subcore_barrier()
    pltpu.sync_copy(acc_sh.at[pl.ds(s * RPT, RPT)],
                    out_hbm.at[c].at[pl.ds(s * RPT, RPT)])


@functools.cache
def _sc_agg_call():
    return pl.kernel(
        _sc_agg_body,
        out_type=jax.ShapeDtypeStruct((NC, NPAD, D), jnp.float32),
        mesh=plsc.VectorSubcoreMesh(core_axis_name="c", subcore_axis_name="s"),
        scratch_types=[
            pltpu.VMEM((GCH, CHUNK), jnp.int32),
            pltpu.VMEM((GCH, CHUNK), jnp.int32),
            pltpu.VMEM((NBUF, CHUNK, D), jnp.float32),
            pltpu.VMEM_SHARED((NPAD, D), jnp.float32),
        ] + [pltpu.SemaphoreType.DMA] * (2 * NBUF),
    )


def kernel(x, edge_index, W0, b0, W1, b1):
    src = edge_index[0].astype(jnp.int32)
    dst = edge_index[1].astype(jnp.int32)
    srcr = jnp.concatenate(
        [src, jnp.arange(EPAD - E, dtype=jnp.int32)]).reshape(NCHUNKS, CHUNK)
    dstr = jnp.concatenate(
        [dst, jnp.full((EPAD - E,), N, jnp.int32)]).reshape(NCHUNKS, CHUNK)
    zeros = jnp.zeros((NPAD, D), jnp.float32)
    W0t = W0.T
    W1t = W1.T
    b0r = b0.reshape(1, D)
    b1r = b1.reshape(1, D)

    sc_agg = _sc_agg_call()
    t0 = _tc_pre(x, W0t, b0r)
    p0 = sc_agg(t0, srcr, dstr, zeros)
    t1 = _tc_mid(p0, W1t, b1r)
    p1 = sc_agg(t1, srcr, dstr, zeros)
    return _tc_post(p1)


# speedup vs baseline: 1.2865x; 1.0261x over previous
import functools

import jax
import jax.numpy as jnp
from jax import lax
from jax.experimental import pallas as pl
from jax.experimental.pallas import tpu as pltpu
from jax.experimental.pallas import tpu_sc as plsc

N = 10000
E = 320000
D = 128

MIN_NORM = 1e-15
MAXNORM = 1.0 - 4e-3

NC, NS, L = 2, 16, 16
NW = NC * NS
CHUNK = 128
NCHUNKS = 2560
EPAD = NCHUNKS * CHUNK
CPT = NCHUNKS // NW
NPAD = 10112
RPT = NPAD // NS

BLK = 2000
GRID = N // BLK


def _rownorm2(x):
    return jnp.sum(x * x, axis=-1, keepdims=True)


def _expmap0(u):
    n = jnp.maximum(jnp.sqrt(_rownorm2(u)), MIN_NORM)
    return jnp.tanh(n) * u / n


def _artanh(x):
    x = jnp.clip(x, -1.0 + 1e-7, 1.0 - 1e-7)
    return 0.5 * jnp.log((1.0 + x) / (1.0 - x))


def _logmap0(p):
    n = jnp.maximum(jnp.sqrt(_rownorm2(p)), MIN_NORM)
    return _artanh(n) * p / n


def _proj(x):
    n = jnp.maximum(jnp.sqrt(_rownorm2(x)), MIN_NORM)
    return jnp.where(n > MAXNORM, x / n * MAXNORM, x)


def _mobius_add(x, y):
    x2 = _rownorm2(x)
    y2 = _rownorm2(y)
    xy = jnp.sum(x * y, axis=-1, keepdims=True)
    num = (1.0 + 2.0 * xy + y2) * x + (1.0 - x2) * y
    denom = 1.0 + 2.0 * xy + x2 * y2
    return num / jnp.maximum(denom, MIN_NORM)


def _mobius_matvec(Wt, x):
    x2s = _rownorm2(x)
    xn = jnp.maximum(jnp.sqrt(x2s), MIN_NORM)
    mx = jnp.dot(x, Wt, preferred_element_type=jnp.float32)
    mx2s = _rownorm2(mx)
    mxn = jnp.maximum(jnp.sqrt(mx2s), MIN_NORM)
    res = jnp.tanh(mxn / xn * _artanh(xn)) * mx / mxn
    return jnp.where(mx2s == 0.0, jnp.zeros_like(res), res)


def _hyp_linear(x, Wt, b):
    res = _proj(_mobius_matvec(Wt, x))
    hyp_bias = _proj(_expmap0(b))
    return _proj(_mobius_add(res, hyp_bias))


def _tc_pre_body(x_ref, w_ref, b_ref, o_ref):
    xh = _proj(_expmap0(x_ref[...]))
    h = _hyp_linear(xh, w_ref[...], b_ref[...])
    o_ref[...] = _logmap0(h)


def _tc_mid_body(p_ref, w_ref, b_ref, o_ref):
    s = p_ref[0] + p_ref[1]
    h = _proj(_expmap0(s))
    h = _proj(_expmap0(jnp.maximum(_logmap0(h), 0.0)))
    h = _hyp_linear(h, w_ref[...], b_ref[...])
    o_ref[...] = _logmap0(h)


def _tc_post_body(p_ref, o_ref):
    s = p_ref[0] + p_ref[1]
    h = _proj(_expmap0(s))
    o_ref[...] = _proj(_expmap0(jnp.maximum(_logmap0(h), 0.0)))


_row_spec = pl.BlockSpec((BLK, D), lambda i: (i, 0))
_par_spec = pl.BlockSpec((2, BLK, D), lambda i: (0, i, 0))
_w_spec = pl.BlockSpec((D, D), lambda i: (0, 0))
_b_spec = pl.BlockSpec((1, D), lambda i: (0, 0))
_out_sd = jax.ShapeDtypeStruct((N, D), jnp.float32)

_tc_pre = pl.pallas_call(
    _tc_pre_body, grid=(GRID,),
    in_specs=[_row_spec, _w_spec, _b_spec], out_specs=_row_spec,
    out_shape=_out_sd)

_tc_mid = pl.pallas_call(
    _tc_mid_body, grid=(GRID,),
    in_specs=[_par_spec, _w_spec, _b_spec], out_specs=_row_spec,
    out_shape=_out_sd)

_tc_post = pl.pallas_call(
    _tc_post_body, grid=(GRID,),
    in_specs=[_par_spec], out_specs=_row_spec,
    out_shape=_out_sd)


NBUF = 2
GCH = 40
CPT0 = 2 * GCH
CPT1 = 2 * GCH


def _run_edges(t_hbm, srcr_hbm, dstr_hbm, src_v, dst_v, rows_v, acc_sh,
               gsems, base, ngroups):
    for g in range(ngroups):
        pltpu.sync_copy(srcr_hbm.at[pl.ds(base + g * GCH, GCH)], src_v)
        pltpu.sync_copy(dstr_hbm.at[pl.ds(base + g * GCH, GCH)], dst_v)
        for b in range(NBUF):
            pltpu.async_copy(t_hbm.at[src_v.at[b]], rows_v.at[b], gsems[b])

        def body(i, carry):
            for b in range(NBUF):
                j = i * NBUF + b
                pltpu.make_async_copy(
                    t_hbm.at[pl.ds(0, CHUNK)], rows_v.at[b], gsems[b]).wait()
                pltpu.sync_copy(rows_v.at[b], acc_sh.at[dst_v.at[j]],
                                add=True)

                @pl.when(j + NBUF < GCH)
                def _():
                    pltpu.async_copy(
                        t_hbm.at[src_v.at[j + NBUF]], rows_v.at[b], gsems[b])
            return carry

        lax.fori_loop(0, GCH // NBUF, body, 0)


def _sc_agg_body(t_hbm, srcr_hbm, dstr_hbm, out_hbm,
                 src_v, dst_v, rows_v, acc_sh, *gsems):
    c = lax.axis_index("c")
    s = lax.axis_index("s")

    def zrow(r, carry):
        for k in range(D // L):
            rows_v[0, r, pl.ds(k * L, L)] = jnp.zeros((L,), jnp.float32)
        return carry

    lax.fori_loop(0, CHUNK, zrow, 0)
    for k in range(RPT // CHUNK):
        pltpu.sync_copy(rows_v.at[0],
                        acc_sh.at[pl.ds(s * RPT + k * CHUNK, CHUNK)])
    _TAIL = RPT - (RPT // CHUNK) * CHUNK
    pltpu.sync_copy(rows_v.at[0].at[pl.ds(0, _TAIL)],
                    acc_sh.at[pl.ds(s * RPT + RPT - _TAIL, _TAIL)])
    plsc.subcore_barrier()

    @pl.when(c == 0)
    def _():
        _run_edges(t_hbm, srcr_hbm, dstr_hbm, src_v, dst_v, rows_v, acc_sh,
                   gsems, s * CPT0, CPT0 // GCH)

    @pl.when(c == 1)
    def _():
        _run_edges(t_hbm, srcr_hbm, dstr_hbm, src_v, dst_v, rows_v, acc_sh,
                   gsems, NS * CPT0 + s * CPT1, CPT1 // GCH)

    plsc.subcore_barrier()
    pltpu.sync_copy(acc_sh.at[pl.ds(s * RPT, RPT)],
                    out_hbm.at[c].at[pl.ds(s * RPT, RPT)])


@functools.cache
def _sc_agg_call():
    return pl.kernel(
        _sc_agg_body,
        out_type=jax.ShapeDtypeStruct((NC, NPAD, D), jnp.float32),
        mesh=plsc.VectorSubcoreMesh(core_axis_name="c", subcore_axis_name="s"),
        scratch_types=[
            pltpu.VMEM((GCH, CHUNK), jnp.int32),
            pltpu.VMEM((GCH, CHUNK), jnp.int32),
            pltpu.VMEM((NBUF, CHUNK, D), jnp.float32),
            pltpu.VMEM_SHARED((NPAD, D), jnp.float32),
        ] + [pltpu.SemaphoreType.DMA] * NBUF,
    )


def kernel(x, edge_index, W0, b0, W1, b1):
    src = edge_index[0].astype(jnp.int32)
    dst = edge_index[1].astype(jnp.int32)
    srcr = jnp.concatenate(
        [src, jnp.arange(EPAD - E, dtype=jnp.int32)]).reshape(NCHUNKS, CHUNK)
    dstr = jnp.concatenate(
        [dst, jnp.full((EPAD - E,), N, jnp.int32)]).reshape(NCHUNKS, CHUNK)
    W0t = W0.T
    W1t = W1.T
    b0r = b0.reshape(1, D)
    b1r = b1.reshape(1, D)

    sc_agg = _sc_agg_call()
    t0 = _tc_pre(x, W0t, b0r)
    p0 = sc_agg(t0, srcr, dstr)
    t1 = _tc_mid(p0, W1t, b1r)
    p1 = sc_agg(t1, srcr, dstr)
    return _tc_post(p1)
